# baseline (speedup 1.0000x reference)
import functools

import jax
import jax.numpy as jnp
from jax import lax
from jax.experimental import pallas as pl
from jax.experimental.pallas import tpu as pltpu
from jax.experimental.pallas import tpu_sc as plsc

_NC = 2
_NS = 16
_NW = _NC * _NS
_K = 128
_BLK = 10


def _sc_aggregate(table, n_rows, srcs, dsts, divs=None, pats=None,
                  pattern=None):
  width = table.shape[1]
  nw, ch, _, k = srcs.shape
  with_deg = pattern is not None
  npt = n_rows // _NS
  dpt = npt // 8
  nblk = ch // _BLK
  mesh = plsc.VectorSubcoreMesh(core_axis_name="c", subcore_axis_name="s")

  out_type = [jax.ShapeDtypeStruct((_NC, n_rows, width), jnp.float32)]
  scratch = [
      pltpu.VMEM((_BLK, 1, k), jnp.int32),
      pltpu.VMEM((_BLK, 1, k), jnp.int32),
      pltpu.VMEM((2, k, width), jnp.float32),
      pltpu.VMEM_SHARED((n_rows, width), jnp.float32),
      pltpu.SemaphoreType.DMA,
      pltpu.SemaphoreType.DMA,
  ]
  if with_deg:
    out_type.append(jax.ShapeDtypeStruct((_NC, n_rows // 8, 128),
                                         jnp.float32))
    scratch.append(pltpu.VMEM((_BLK, 1, k), jnp.int32))
    scratch.append(pltpu.VMEM((_BLK, 1, k), jnp.int32))
    scratch.append(pltpu.VMEM_SHARED((n_rows // 8, 128), jnp.float32))

  def body(table_hbm, *rest):
    if with_deg:
      (src_hbm, dst_hbm, div_hbm, pat_hbm, ptab_hbm, sums_out, deg_out,
       src_b, dst_b, rows_v, sums_sh, sem_g, sem_s, div_b, pat_b,
       deg_sh) = rest
    else:
      (src_hbm, dst_hbm, sums_out, src_b, dst_b, rows_v, sums_sh, sem_g,
       sem_s) = rest
    c = lax.axis_index("c")
    s = lax.axis_index("s")
    wid = c * _NS + s

    def zrow(i, _):
      for cc in range(width // 16):
        rows_v[0, i, pl.ds(cc * 16, 16)] = jnp.zeros((16,), jnp.float32)
      return 0
    lax.fori_loop(0, k, zrow, 0)
    base = s * npt
    for t in range(npt // k):
      pltpu.sync_copy(rows_v.at[0], sums_sh.at[pl.ds(base + t * k, k)])
    if with_deg:
      pltpu.sync_copy(rows_v.at[0, pl.ds(0, dpt)],
                      deg_sh.at[pl.ds(s * dpt, dpt)])
    plsc.subcore_barrier()

    def run_pipe(mk_g, mk_s):
      mk_g(0, 0).start()

      def step(j, _):
        @pl.when(j >= 1)
        def _():
          mk_s(j - 1, (j - 1) % 2).wait()

        @pl.when(j < _BLK - 1)
        def _():
          mk_g(j + 1, (j + 1) % 2).start()
        mk_g(j, j % 2).wait()
        mk_s(j, j % 2).start(add=True)
        return 0
      lax.fori_loop(0, _BLK, step, 0)
      mk_s(_BLK - 1, (_BLK - 1) % 2).wait()

    def mk_sum_g(j, b):
      return pltpu.make_async_copy(
          table_hbm.at[src_b.at[j, 0]], rows_v.at[b], sem_g)

    def mk_sum_s(j, b):
      return pltpu.make_async_copy(
          rows_v.at[b], sums_sh.at[dst_b.at[j, 0]], sem_s)

    def block_step(bi, _):
      sl = pl.ds(bi * _BLK, _BLK)
      pltpu.sync_copy(src_hbm.at[wid, sl], src_b)
      pltpu.sync_copy(dst_hbm.at[wid, sl], dst_b)
      run_pipe(mk_sum_g, mk_sum_s)
      if with_deg:
        def mk_deg_g(j, b):
          return pltpu.make_async_copy(
              ptab_hbm.at[pat_b.at[j, 0]], rows_v.at[b], sem_g)

        def mk_deg_s(j, b):
          return pltpu.make_async_copy(
              rows_v.at[b], deg_sh.at[div_b.at[j, 0]], sem_s)
        pltpu.sync_copy(div_hbm.at[wid, sl], div_b)
        pltpu.sync_copy(pat_hbm.at[wid, sl], pat_b)
        run_pipe(mk_deg_g, mk_deg_s)
      return 0
    lax.fori_loop(0, nblk, block_step, 0)

    plsc.subcore_barrier()
    pltpu.sync_copy(sums_sh.at[pl.ds(base, npt)],
                    sums_out.at[c, pl.ds(base, npt)])
    if with_deg:
      pltpu.sync_copy(deg_sh.at[pl.ds(s * dpt, dpt)],
                      deg_out.at[c, pl.ds(s * dpt, dpt)])

  fn = pl.kernel(body, out_type=out_type, mesh=mesh, scratch_types=scratch)
  if with_deg:
    return fn(table, srcs, dsts, divs, pats, pattern)
  return fn(table, srcs, dsts)


def _unfold_deg(dp):
  return jnp.maximum(dp[0, :, :1] + dp[1, :, :1], 1.0)


def _tc_layer1(features, sums_p, deg_p, W_self1, W_neigh1, b1,
               W_self2, b2):
  n, d = features.shape
  h = W_self1.shape[1]
  c_dim = W_self2.shape[1]
  r = 1024
  nb = (n + r - 1) // r

  def body(f_ref, sp_ref, dp_ref, ws1_ref, wn1_ref, b1_ref, ws2_ref,
           b2_ref, h1_ref, pre2_ref):
    deg = _unfold_deg(dp_ref[...])
    sp = sp_ref[...]
    sums1 = sp[0] + sp[1]
    hn1 = jnp.dot(sums1, wn1_ref[...],
                  preferred_element_type=jnp.float32) / deg
    h1 = jnp.maximum(
        jnp.dot(f_ref[...], ws1_ref[...], preferred_element_type=jnp.float32)
        + hn1 + b1_ref[...], 0.0)
    h1_ref[...] = h1
    pre2_ref[...] = (
        jnp.dot(h1, ws2_ref[...], preferred_element_type=jnp.float32)
        + b2_ref[...])

  return pl.pallas_call(
      body,
      grid=(nb,),
      in_specs=[
          pl.BlockSpec((r, d), lambda i: (i, 0)),
          pl.BlockSpec((2, r, d), lambda i: (0, i, 0)),
          pl.BlockSpec((2, r, 16), lambda i: (0, i, 0)),
          pl.BlockSpec((d, h), lambda i: (0, 0)),
          pl.BlockSpec((d, h), lambda i: (0, 0)),
          pl.BlockSpec((1, h), lambda i: (0, 0)),
          pl.BlockSpec((h, c_dim), lambda i: (0, 0)),
          pl.BlockSpec((1, c_dim), lambda i: (0, 0)),
      ],
      out_specs=[
          pl.BlockSpec((r, h), lambda i: (i, 0)),
          pl.BlockSpec((r, c_dim), lambda i: (i, 0)),
      ],
      out_shape=[
          jax.ShapeDtypeStruct((n, h), jnp.float32),
          jax.ShapeDtypeStruct((n, c_dim), jnp.float32),
      ],
  )(features, sums_p, deg_p, W_self1, W_neigh1, b1.reshape(1, h),
    W_self2, b2.reshape(1, c_dim))


def _tc_layer2(pre2, sums2_p, deg_p, W_neigh2):
  n, c_dim = pre2.shape
  h = W_neigh2.shape[0]
  r = 1024
  nb = (n + r - 1) // r

  def body(pre_ref, q_ref, dp_ref, wn2_ref, out_ref):
    deg = _unfold_deg(dp_ref[...])
    q = q_ref[...]
    out_ref[...] = pre_ref[...] + jnp.dot(
        q[0] + q[1], wn2_ref[...], preferred_element_type=jnp.float32) / deg

  return pl.pallas_call(
      body,
      grid=(nb,),
      in_specs=[
          pl.BlockSpec((r, c_dim), lambda i: (i, 0)),
          pl.BlockSpec((2, r, h), lambda i: (0, i, 0)),
          pl.BlockSpec((2, r, 16), lambda i: (0, i, 0)),
          pl.BlockSpec((h, c_dim), lambda i: (0, 0)),
      ],
      out_specs=pl.BlockSpec((r, c_dim), lambda i: (i, 0)),
      out_shape=jax.ShapeDtypeStruct((n, c_dim), jnp.float32),
  )(pre2, sums2_p, deg_p, W_neigh2)


@jax.jit
def kernel(features, edge_index, W_self1, W_neigh1, b1, W_self2, W_neigh2,
           b2):
  n = features.shape[0]
  e = edge_index.shape[1]
  kpc = _NW * _K
  ch = -(-(-(-e // kpc)) // _BLK) * _BLK
  e_pad = ch * kpc
  src = edge_index[0]
  dst = edge_index[1]
  if e_pad != e:
    src = jnp.concatenate([src, jnp.zeros((e_pad - e,), jnp.int32)])
    dst = jnp.concatenate([dst, jnp.full((e_pad - e,), n, jnp.int32)])
  pattern = jnp.tile(
      jnp.kron(jnp.eye(8, dtype=jnp.float32),
               jnp.ones((1, 16), jnp.float32)), (256, 1))
  pat_idx = (dst & 7) + 8 * (jnp.arange(e_pad, dtype=jnp.int32) & 255)
  shape3 = (_NW, ch, 1, _K)
  srcs = src.reshape(shape3)
  dsts = dst.reshape(shape3)
  divs = (dst >> 3).reshape(shape3)
  pats = pat_idx.reshape(shape3)
  np_rows = -(-n // (_NS * _K)) * (_NS * _K)

  sums1_p, deg_p = _sc_aggregate(features, np_rows, srcs, dsts,
                                 divs=divs, pats=pats, pattern=pattern)
  deg_p = deg_p.reshape(_NC, np_rows, 16)
  h1, pre2 = _tc_layer1(features, sums1_p, deg_p, W_self1, W_neigh1, b1,
                        W_self2, b2)
  (sums2_p,) = _sc_aggregate(h1, np_rows, srcs, dsts)
  return _tc_layer2(pre2, sums2_p, deg_p, W_neigh2)


# speedup vs baseline: 2.5382x; 2.5382x over previous
import functools

import jax
import jax.numpy as jnp
from jax import lax
from jax.experimental import pallas as pl
from jax.experimental.pallas import tpu as pltpu
from jax.experimental.pallas import tpu_sc as plsc

_NC = 2
_NS = 16
_NW = _NC * _NS
_K = 80
_BLK = 25


def _sc_aggregate(table, n_rows, srcs, dsts, divs=None, pats=None,
                  pattern=None):
  width = table.shape[1]
  nw, ch, _, k = srcs.shape
  with_deg = pattern is not None
  npt = n_rows // _NS
  dpt = npt // 8
  nblk = ch // _BLK
  mesh = plsc.VectorSubcoreMesh(core_axis_name="c", subcore_axis_name="s")

  out_type = [jax.ShapeDtypeStruct((_NC, n_rows, width), jnp.float32)]
  scratch = [
      pltpu.VMEM((_BLK, 1, k), jnp.int32),
      pltpu.VMEM((_BLK, 1, k), jnp.int32),
      pltpu.VMEM((2, k, width), jnp.float32),
      pltpu.VMEM_SHARED((n_rows, width), jnp.float32),
      pltpu.SemaphoreType.DMA,
      pltpu.SemaphoreType.DMA,
  ]
  if with_deg:
    out_type.append(jax.ShapeDtypeStruct((_NC, n_rows // 8, 128),
                                         jnp.float32))
    scratch.append(pltpu.VMEM((_BLK, 1, k), jnp.int32))
    scratch.append(pltpu.VMEM((_BLK, 1, k), jnp.int32))
    scratch.append(pltpu.VMEM_SHARED((n_rows // 8, 128), jnp.float32))

  def body(table_hbm, *rest):
    if with_deg:
      (src_hbm, dst_hbm, div_hbm, pat_hbm, ptab_hbm, sums_out, deg_out,
       src_b, dst_b, rows_v, sums_sh, sem_g, sem_s, div_b, pat_b,
       deg_sh) = rest
    else:
      (src_hbm, dst_hbm, sums_out, src_b, dst_b, rows_v, sums_sh, sem_g,
       sem_s) = rest
    c = lax.axis_index("c")
    s = lax.axis_index("s")
    wid = c * _NS + s

    def zrow(i, _):
      for cc in range(width // 16):
        rows_v[0, i, pl.ds(cc * 16, 16)] = jnp.zeros((16,), jnp.float32)
      return 0
    lax.fori_loop(0, k, zrow, 0)
    base = s * npt
    for t in range(npt // k):
      pltpu.sync_copy(rows_v.at[0], sums_sh.at[pl.ds(base + t * k, k)])
    if with_deg:
      pltpu.sync_copy(rows_v.at[0, pl.ds(0, dpt)],
                      deg_sh.at[pl.ds(s * dpt, dpt)])
    plsc.subcore_barrier()

    def run_pipe(mk_g, mk_s):
      mk_g(0, 0).start()

      def step(j, _):
        @pl.when(j >= 1)
        def _():
          mk_s(j - 1, (j - 1) % 2).wait()

        @pl.when(j < _BLK - 1)
        def _():
          mk_g(j + 1, (j + 1) % 2).start()
        mk_g(j, j % 2).wait()
        mk_s(j, j % 2).start(add=True)
        return 0
      lax.fori_loop(0, _BLK, step, 0)
      mk_s(_BLK - 1, (_BLK - 1) % 2).wait()

    def mk_sum_g(j, b):
      return pltpu.make_async_copy(
          table_hbm.at[src_b.at[j, 0]], rows_v.at[b], sem_g)

    def mk_sum_s(j, b):
      return pltpu.make_async_copy(
          rows_v.at[b], sums_sh.at[dst_b.at[j, 0]], sem_s)

    def block_step(bi, _):
      sl = pl.ds(bi * _BLK, _BLK)
      pltpu.sync_copy(src_hbm.at[wid, sl], src_b)
      pltpu.sync_copy(dst_hbm.at[wid, sl], dst_b)
      run_pipe(mk_sum_g, mk_sum_s)
      if with_deg:
        def mk_deg_g(j, b):
          return pltpu.make_async_copy(
              ptab_hbm.at[pat_b.at[j, 0]], rows_v.at[b], sem_g)

        def mk_deg_s(j, b):
          return pltpu.make_async_copy(
              rows_v.at[b], deg_sh.at[div_b.at[j, 0]], sem_s)
        pltpu.sync_copy(div_hbm.at[wid, sl], div_b)
        pltpu.sync_copy(pat_hbm.at[wid, sl], pat_b)
        run_pipe(mk_deg_g, mk_deg_s)
      return 0
    lax.fori_loop(0, nblk, block_step, 0)

    plsc.subcore_barrier()
    pltpu.sync_copy(sums_sh.at[pl.ds(base, npt)],
                    sums_out.at[c, pl.ds(base, npt)])
    if with_deg:
      pltpu.sync_copy(deg_sh.at[pl.ds(s * dpt, dpt)],
                      deg_out.at[c, pl.ds(s * dpt, dpt)])

  fn = pl.kernel(body, out_type=out_type, mesh=mesh, scratch_types=scratch)
  if with_deg:
    return fn(table, srcs, dsts, divs, pats, pattern)
  return fn(table, srcs, dsts)


def _unfold_deg(dp):
  return jnp.maximum(dp[0, :, :1] + dp[1, :, :1], 1.0)


def _tc_layer1(features, sums_p, deg_p, W_self1, W_neigh1, b1,
               W_self2, b2):
  n, d = features.shape
  h = W_self1.shape[1]
  c_dim = W_self2.shape[1]
  r = 1024
  nb = (n + r - 1) // r

  def body(f_ref, sp_ref, dp_ref, ws1_ref, wn1_ref, b1_ref, ws2_ref,
           b2_ref, h1_ref, pre2_ref):
    deg = _unfold_deg(dp_ref[...])
    sp = sp_ref[...]
    sums1 = sp[0] + sp[1]
    hn1 = jnp.dot(sums1, wn1_ref[...],
                  preferred_element_type=jnp.float32) / deg
    h1 = jnp.maximum(
        jnp.dot(f_ref[...], ws1_ref[...], preferred_element_type=jnp.float32)
        + hn1 + b1_ref[...], 0.0)
    h1_ref[...] = h1
    pre2_ref[...] = (
        jnp.dot(h1, ws2_ref[...], preferred_element_type=jnp.float32)
        + b2_ref[...])

  return pl.pallas_call(
      body,
      grid=(nb,),
      in_specs=[
          pl.BlockSpec((r, d), lambda i: (i, 0)),
          pl.BlockSpec((2, r, d), lambda i: (0, i, 0)),
          pl.BlockSpec((2, r, 16), lambda i: (0, i, 0)),
          pl.BlockSpec((d, h), lambda i: (0, 0)),
          pl.BlockSpec((d, h), lambda i: (0, 0)),
          pl.BlockSpec((1, h), lambda i: (0, 0)),
          pl.BlockSpec((h, c_dim), lambda i: (0, 0)),
          pl.BlockSpec((1, c_dim), lambda i: (0, 0)),
      ],
      out_specs=[
          pl.BlockSpec((r, h), lambda i: (i, 0)),
          pl.BlockSpec((r, c_dim), lambda i: (i, 0)),
      ],
      out_shape=[
          jax.ShapeDtypeStruct((n, h), jnp.float32),
          jax.ShapeDtypeStruct((n, c_dim), jnp.float32),
      ],
  )(features, sums_p, deg_p, W_self1, W_neigh1, b1.reshape(1, h),
    W_self2, b2.reshape(1, c_dim))


def _tc_layer2(pre2, sums2_p, deg_p, W_neigh2):
  n, c_dim = pre2.shape
  h = W_neigh2.shape[0]
  r = 1024
  nb = (n + r - 1) // r

  def body(pre_ref, q_ref, dp_ref, wn2_ref, out_ref):
    deg = _unfold_deg(dp_ref[...])
    q = q_ref[...]
    out_ref[...] = pre_ref[...] + jnp.dot(
        q[0] + q[1], wn2_ref[...], preferred_element_type=jnp.float32) / deg

  return pl.pallas_call(
      body,
      grid=(nb,),
      in_specs=[
          pl.BlockSpec((r, c_dim), lambda i: (i, 0)),
          pl.BlockSpec((2, r, h), lambda i: (0, i, 0)),
          pl.BlockSpec((2, r, 16), lambda i: (0, i, 0)),
          pl.BlockSpec((h, c_dim), lambda i: (0, 0)),
      ],
      out_specs=pl.BlockSpec((r, c_dim), lambda i: (i, 0)),
      out_shape=jax.ShapeDtypeStruct((n, c_dim), jnp.float32),
  )(pre2, sums2_p, deg_p, W_neigh2)


@jax.jit
def kernel(features, edge_index, W_self1, W_neigh1, b1, W_self2, W_neigh2,
           b2):
  n = features.shape[0]
  e = edge_index.shape[1]
  kpc = _NW * _K
  ch = -(-(-(-e // kpc)) // _BLK) * _BLK
  e_pad = ch * kpc
  src = edge_index[0]
  dst = edge_index[1]
  if e_pad != e:
    src = jnp.concatenate([src, jnp.zeros((e_pad - e,), jnp.int32)])
    dst = jnp.concatenate([dst, jnp.full((e_pad - e,), n, jnp.int32)])
  pattern = jnp.tile(
      jnp.kron(jnp.eye(8, dtype=jnp.float32),
               jnp.ones((1, 16), jnp.float32)), (256, 1))
  pat_idx = (dst & 7) + 8 * (jnp.arange(e_pad, dtype=jnp.int32) & 255)
  shape3 = (_NW, ch, 1, _K)
  srcs = src.reshape(shape3)
  dsts = dst.reshape(shape3)
  divs = (dst >> 3).reshape(shape3)
  pats = pat_idx.reshape(shape3)
  np_rows = -(-n // (_NS * _K)) * (_NS * _K)

  sums1_p, deg_p = _sc_aggregate(features, np_rows, srcs, dsts,
                                 divs=divs, pats=pats, pattern=pattern)
  deg_p = deg_p.reshape(_NC, np_rows, 16)
  h1, pre2 = _tc_layer1(features, sums1_p, deg_p, W_self1, W_neigh1, b1,
                        W_self2, b2)
  (sums2_p,) = _sc_aggregate(h1, np_rows, srcs, dsts)
  return _tc_layer2(pre2, sums2_p, deg_p, W_neigh2)


# speedup vs baseline: 2.6695x; 1.0517x over previous
import functools

import jax
import jax.numpy as jnp
from jax import lax
from jax.experimental import pallas as pl
from jax.experimental.pallas import tpu as pltpu
from jax.experimental.pallas import tpu_sc as plsc

_NC = 2
_NS = 16
_NW = _NC * _NS
_K = 80
_BLK = 25


def _sc_aggregate(table, n_rows, srcs, dsts, divs=None, pats=None,
                  pattern=None):
  width = table.shape[1]
  nw, ch, _, k = srcs.shape
  with_deg = pattern is not None
  npt = n_rows // _NS
  dpt = npt // 8
  nblk = ch // _BLK
  mesh = plsc.VectorSubcoreMesh(core_axis_name="c", subcore_axis_name="s")

  out_type = [jax.ShapeDtypeStruct((_NC, n_rows, width), jnp.float32)]
  scratch = [
      pltpu.VMEM((_BLK, 1, k), jnp.int32),
      pltpu.VMEM((_BLK, 1, k), jnp.int32),
      pltpu.VMEM((2 if with_deg else 3, k, width), jnp.float32),
      pltpu.VMEM_SHARED((n_rows, width), jnp.float32),
      pltpu.SemaphoreType.DMA,
      pltpu.SemaphoreType.DMA,
  ]
  if with_deg:
    out_type.append(jax.ShapeDtypeStruct((_NC, n_rows // 8, 128),
                                         jnp.float32))
    scratch.append(pltpu.VMEM((_BLK, 1, k), jnp.int32))
    scratch.append(pltpu.VMEM((_BLK, 1, k), jnp.int32))
    scratch.append(pltpu.VMEM_SHARED((n_rows // 8, 128), jnp.float32))

  def body(table_hbm, *rest):
    if with_deg:
      (src_hbm, dst_hbm, div_hbm, pat_hbm, ptab_hbm, sums_out, deg_out,
       src_b, dst_b, rows_v, sums_sh, sem_g, sem_s, div_b, pat_b,
       deg_sh) = rest
    else:
      (src_hbm, dst_hbm, sums_out, src_b, dst_b, rows_v, sums_sh, sem_g,
       sem_s) = rest
    c = lax.axis_index("c")
    s = lax.axis_index("s")
    wid = c * _NS + s

    def zrow(i, _):
      for cc in range(width // 16):
        rows_v[0, i, pl.ds(cc * 16, 16)] = jnp.zeros((16,), jnp.float32)
      return 0
    lax.fori_loop(0, k, zrow, 0)
    base = s * npt
    for t in range(npt // k):
      pltpu.sync_copy(rows_v.at[0], sums_sh.at[pl.ds(base + t * k, k)])
    if with_deg:
      pltpu.sync_copy(rows_v.at[0, pl.ds(0, dpt)],
                      deg_sh.at[pl.ds(s * dpt, dpt)])
    plsc.subcore_barrier()

    nbuf = 2 if with_deg else 3

    def run_pipe(mk_g, mk_s):
      for i in range(nbuf - 1):
        mk_g(i, i).start()

      def step(j, _):
        @pl.when(j >= 1)
        def _():
          mk_s(j - 1, (j - 1) % nbuf).wait()

        @pl.when(j < _BLK - nbuf + 1)
        def _():
          mk_g(j + nbuf - 1, (j + nbuf - 1) % nbuf).start()
        mk_g(j, j % nbuf).wait()
        mk_s(j, j % nbuf).start(add=True)
        return 0
      lax.fori_loop(0, _BLK, step, 0)
      mk_s(_BLK - 1, (_BLK - 1) % nbuf).wait()

    def mk_sum_g(j, b):
      return pltpu.make_async_copy(
          table_hbm.at[src_b.at[j, 0]], rows_v.at[b], sem_g)

    def mk_sum_s(j, b):
      return pltpu.make_async_copy(
          rows_v.at[b], sums_sh.at[dst_b.at[j, 0]], sem_s)

    def block_step(bi, _):
      sl = pl.ds(bi * _BLK, _BLK)
      pltpu.sync_copy(src_hbm.at[wid, sl], src_b)
      pltpu.sync_copy(dst_hbm.at[wid, sl], dst_b)
      run_pipe(mk_sum_g, mk_sum_s)
      if with_deg:
        def mk_deg_g(j, b):
          return pltpu.make_async_copy(
              ptab_hbm.at[pat_b.at[j, 0]], rows_v.at[b], sem_g)

        def mk_deg_s(j, b):
          return pltpu.make_async_copy(
              rows_v.at[b], deg_sh.at[div_b.at[j, 0]], sem_s)
        pltpu.sync_copy(div_hbm.at[wid, sl], div_b)
        pltpu.sync_copy(pat_hbm.at[wid, sl], pat_b)
        run_pipe(mk_deg_g, mk_deg_s)
      return 0
    lax.fori_loop(0, nblk, block_step, 0)

    plsc.subcore_barrier()
    pltpu.sync_copy(sums_sh.at[pl.ds(base, npt)],
                    sums_out.at[c, pl.ds(base, npt)])
    if with_deg:
      pltpu.sync_copy(deg_sh.at[pl.ds(s * dpt, dpt)],
                      deg_out.at[c, pl.ds(s * dpt, dpt)])

  fn = pl.kernel(body, out_type=out_type, mesh=mesh, scratch_types=scratch)
  if with_deg:
    return fn(table, srcs, dsts, divs, pats, pattern)
  return fn(table, srcs, dsts)


def _unfold_deg(dp):
  return jnp.maximum(dp[0, :, :1] + dp[1, :, :1], 1.0)


def _tc_layer1(features, sums_p, deg_p, W_self1, W_neigh1, b1,
               W_self2, b2):
  n, d = features.shape
  h = W_self1.shape[1]
  c_dim = W_self2.shape[1]
  r = 1024
  nb = (n + r - 1) // r

  def body(f_ref, sp_ref, dp_ref, ws1_ref, wn1_ref, b1_ref, ws2_ref,
           b2_ref, h1_ref, pre2_ref):
    deg = _unfold_deg(dp_ref[...])
    sp = sp_ref[...]
    sums1 = sp[0] + sp[1]
    hn1 = jnp.dot(sums1, wn1_ref[...],
                  preferred_element_type=jnp.float32) / deg
    h1 = jnp.maximum(
        jnp.dot(f_ref[...], ws1_ref[...], preferred_element_type=jnp.float32)
        + hn1 + b1_ref[...], 0.0)
    h1_ref[...] = h1
    pre2_ref[...] = (
        jnp.dot(h1, ws2_ref[...], preferred_element_type=jnp.float32)
        + b2_ref[...])

  return pl.pallas_call(
      body,
      grid=(nb,),
      in_specs=[
          pl.BlockSpec((r, d), lambda i: (i, 0)),
          pl.BlockSpec((2, r, d), lambda i: (0, i, 0)),
          pl.BlockSpec((2, r, 16), lambda i: (0, i, 0)),
          pl.BlockSpec((d, h), lambda i: (0, 0)),
          pl.BlockSpec((d, h), lambda i: (0, 0)),
          pl.BlockSpec((1, h), lambda i: (0, 0)),
          pl.BlockSpec((h, c_dim), lambda i: (0, 0)),
          pl.BlockSpec((1, c_dim), lambda i: (0, 0)),
      ],
      out_specs=[
          pl.BlockSpec((r, h), lambda i: (i, 0)),
          pl.BlockSpec((r, c_dim), lambda i: (i, 0)),
      ],
      out_shape=[
          jax.ShapeDtypeStruct((n, h), jnp.float32),
          jax.ShapeDtypeStruct((n, c_dim), jnp.float32),
      ],
  )(features, sums_p, deg_p, W_self1, W_neigh1, b1.reshape(1, h),
    W_self2, b2.reshape(1, c_dim))


def _tc_layer2(pre2, sums2_p, deg_p, W_neigh2):
  n, c_dim = pre2.shape
  h = W_neigh2.shape[0]
  r = 1024
  nb = (n + r - 1) // r

  def body(pre_ref, q_ref, dp_ref, wn2_ref, out_ref):
    deg = _unfold_deg(dp_ref[...])
    q = q_ref[...]
    out_ref[...] = pre_ref[...] + jnp.dot(
        q[0] + q[1], wn2_ref[...], preferred_element_type=jnp.float32) / deg

  return pl.pallas_call(
      body,
      grid=(nb,),
      in_specs=[
          pl.BlockSpec((r, c_dim), lambda i: (i, 0)),
          pl.BlockSpec((2, r, h), lambda i: (0, i, 0)),
          pl.BlockSpec((2, r, 16), lambda i: (0, i, 0)),
          pl.BlockSpec((h, c_dim), lambda i: (0, 0)),
      ],
      out_specs=pl.BlockSpec((r, c_dim), lambda i: (i, 0)),
      out_shape=jax.ShapeDtypeStruct((n, c_dim), jnp.float32),
  )(pre2, sums2_p, deg_p, W_neigh2)


@jax.jit
def kernel(features, edge_index, W_self1, W_neigh1, b1, W_self2, W_neigh2,
           b2):
  n = features.shape[0]
  e = edge_index.shape[1]
  kpc = _NW * _K
  ch = -(-(-(-e // kpc)) // _BLK) * _BLK
  e_pad = ch * kpc
  src = edge_index[0]
  dst = edge_index[1]
  if e_pad != e:
    src = jnp.concatenate([src, jnp.zeros((e_pad - e,), jnp.int32)])
    dst = jnp.concatenate([dst, jnp.full((e_pad - e,), n, jnp.int32)])
  pattern = jnp.tile(
      jnp.kron(jnp.eye(8, dtype=jnp.float32),
               jnp.ones((1, 16), jnp.float32)), (256, 1))
  pat_idx = (dst & 7) + 8 * (jnp.arange(e_pad, dtype=jnp.int32) & 255)
  shape3 = (_NW, ch, 1, _K)
  srcs = src.reshape(shape3)
  dsts = dst.reshape(shape3)
  divs = (dst >> 3).reshape(shape3)
  pats = pat_idx.reshape(shape3)
  np_rows = -(-n // (_NS * _K)) * (_NS * _K)

  sums1_p, deg_p = _sc_aggregate(features, np_rows, srcs, dsts,
                                 divs=divs, pats=pats, pattern=pattern)
  deg_p = deg_p.reshape(_NC, np_rows, 16)
  h1, pre2 = _tc_layer1(features, sums1_p, deg_p, W_self1, W_neigh1, b1,
                        W_self2, b2)
  (sums2_p,) = _sc_aggregate(h1, np_rows, srcs, dsts)
  return _tc_layer2(pre2, sums2_p, deg_p, W_neigh2)


# speedup vs baseline: 2.9294x; 1.0974x over previous
import functools

import jax
import jax.numpy as jnp
from jax import lax
from jax.experimental import pallas as pl
from jax.experimental.pallas import tpu as pltpu
from jax.experimental.pallas import tpu_sc as plsc

_NC = 2
_NS = 16
_NW = _NC * _NS
_K = 80
_BLK = 25


def _sc_aggregate(table, n_rows, srcs, dsts, divs=None, pats=None,
                  pattern=None):
  width = table.shape[1]
  nw, ch, _, k = srcs.shape
  with_deg = pattern is not None
  npt = n_rows // _NS
  dpt = npt // 16
  nblk = ch // _BLK
  mesh = plsc.VectorSubcoreMesh(core_axis_name="c", subcore_axis_name="s")

  out_type = [jax.ShapeDtypeStruct((_NC, n_rows, width), jnp.float32)]
  scratch = [
      pltpu.VMEM((_BLK, 1, k), jnp.int32),
      pltpu.VMEM((_BLK, 1, k), jnp.int32),
      pltpu.VMEM((3, k, width), jnp.float32),
      pltpu.VMEM_SHARED((n_rows, width), jnp.float32),
      pltpu.SemaphoreType.DMA,
      pltpu.SemaphoreType.DMA,
  ]
  if with_deg:
    out_type.append(jax.ShapeDtypeStruct((_NC, n_rows // 16, 128),
                                         jnp.float32))
    scratch.append(pltpu.VMEM((_BLK, 1, k), jnp.int32))
    scratch.append(pltpu.VMEM((_BLK, 1, k), jnp.int32))
    scratch.append(pltpu.VMEM_SHARED((n_rows // 16, 128), jnp.float32))

  def body(table_hbm, *rest):
    if with_deg:
      (src_hbm, dst_hbm, div_hbm, pat_hbm, ptab_hbm, sums_out, deg_out,
       src_b, dst_b, rows_v, sums_sh, sem_g, sem_s, div_b, pat_b,
       deg_sh) = rest
    else:
      (src_hbm, dst_hbm, sums_out, src_b, dst_b, rows_v, sums_sh, sem_g,
       sem_s) = rest
    c = lax.axis_index("c")
    s = lax.axis_index("s")
    wid = c * _NS + s

    def zrow(i, _):
      for cc in range(width // 16):
        rows_v[0, i, pl.ds(cc * 16, 16)] = jnp.zeros((16,), jnp.float32)
      return 0
    lax.fori_loop(0, k, zrow, 0)
    base = s * npt
    for t in range(npt // k):
      pltpu.sync_copy(rows_v.at[0], sums_sh.at[pl.ds(base + t * k, k)])
    if with_deg:
      pltpu.sync_copy(rows_v.at[0, pl.ds(0, dpt)],
                      deg_sh.at[pl.ds(s * dpt, dpt)])
    plsc.subcore_barrier()

    nbuf = 3

    def run_pipe(mk_g, mk_s):
      for i in range(nbuf - 1):
        mk_g(i, i).start()

      def step(j, _):
        @pl.when(j >= 1)
        def _():
          mk_s(j - 1, (j - 1) % nbuf).wait()

        @pl.when(j < _BLK - nbuf + 1)
        def _():
          mk_g(j + nbuf - 1, (j + nbuf - 1) % nbuf).start()
        mk_g(j, j % nbuf).wait()
        mk_s(j, j % nbuf).start(add=True)
        return 0
      lax.fori_loop(0, _BLK, step, 0)
      mk_s(_BLK - 1, (_BLK - 1) % nbuf).wait()

    def mk_sum_g(j, b):
      return pltpu.make_async_copy(
          table_hbm.at[src_b.at[j, 0]], rows_v.at[b], sem_g)

    def mk_sum_s(j, b):
      return pltpu.make_async_copy(
          rows_v.at[b], sums_sh.at[dst_b.at[j, 0]], sem_s)

    def block_step(bi, _):
      sl = pl.ds(bi * _BLK, _BLK)
      pltpu.sync_copy(src_hbm.at[wid, sl], src_b)
      pltpu.sync_copy(dst_hbm.at[wid, sl], dst_b)
      run_pipe(mk_sum_g, mk_sum_s)
      if with_deg:
        def mk_deg_g(j, b):
          return pltpu.make_async_copy(
              ptab_hbm.at[pat_b.at[j, 0]], rows_v.at[b], sem_g)

        def mk_deg_s(j, b):
          return pltpu.make_async_copy(
              rows_v.at[b], deg_sh.at[div_b.at[j, 0]], sem_s)
        pltpu.sync_copy(div_hbm.at[wid, sl], div_b)
        pltpu.sync_copy(pat_hbm.at[wid, sl], pat_b)
        run_pipe(mk_deg_g, mk_deg_s)
      return 0
    lax.fori_loop(0, nblk, block_step, 0)

    plsc.subcore_barrier()
    pltpu.sync_copy(sums_sh.at[pl.ds(base, npt)],
                    sums_out.at[c, pl.ds(base, npt)])
    if with_deg:
      pltpu.sync_copy(deg_sh.at[pl.ds(s * dpt, dpt)],
                      deg_out.at[c, pl.ds(s * dpt, dpt)])

  fn = pl.kernel(body, out_type=out_type, mesh=mesh, scratch_types=scratch)
  if with_deg:
    return fn(table, srcs, dsts, divs, pats, pattern)
  return fn(table, srcs, dsts)


def _unfold_deg(dp):
  return jnp.maximum(dp[0, :, :1] + dp[1, :, :1], 1.0)


def _tc_layer1(features, sums_p, deg_p, W_self1, W_neigh1, b1,
               W_self2, b2):
  n, d = features.shape
  h = W_self1.shape[1]
  c_dim = W_self2.shape[1]
  r = 1024
  nb = (n + r - 1) // r

  def body(f_ref, sp_ref, dp_ref, ws1_ref, wn1_ref, b1_ref, ws2_ref,
           b2_ref, h1_ref, pre2_ref):
    deg = _unfold_deg(dp_ref[...])
    sp = sp_ref[...]
    sums1 = sp[0] + sp[1]
    hn1 = jnp.dot(sums1, wn1_ref[...],
                  preferred_element_type=jnp.float32) / deg
    h1 = jnp.maximum(
        jnp.dot(f_ref[...], ws1_ref[...], preferred_element_type=jnp.float32)
        + hn1 + b1_ref[...], 0.0)
    h1_ref[...] = h1
    pre2_ref[...] = (
        jnp.dot(h1, ws2_ref[...], preferred_element_type=jnp.float32)
        + b2_ref[...])

  return pl.pallas_call(
      body,
      grid=(nb,),
      in_specs=[
          pl.BlockSpec((r, d), lambda i: (i, 0)),
          pl.BlockSpec((2, r, d), lambda i: (0, i, 0)),
          pl.BlockSpec((2, r, 8), lambda i: (0, i, 0)),
          pl.BlockSpec((d, h), lambda i: (0, 0)),
          pl.BlockSpec((d, h), lambda i: (0, 0)),
          pl.BlockSpec((1, h), lambda i: (0, 0)),
          pl.BlockSpec((h, c_dim), lambda i: (0, 0)),
          pl.BlockSpec((1, c_dim), lambda i: (0, 0)),
      ],
      out_specs=[
          pl.BlockSpec((r, h), lambda i: (i, 0)),
          pl.BlockSpec((r, c_dim), lambda i: (i, 0)),
      ],
      out_shape=[
          jax.ShapeDtypeStruct((n, h), jnp.float32),
          jax.ShapeDtypeStruct((n, c_dim), jnp.float32),
      ],
  )(features, sums_p, deg_p, W_self1, W_neigh1, b1.reshape(1, h),
    W_self2, b2.reshape(1, c_dim))


def _tc_layer2(pre2, sums2_p, deg_p, W_neigh2):
  n, c_dim = pre2.shape
  h = W_neigh2.shape[0]
  r = 1024
  nb = (n + r - 1) // r

  def body(pre_ref, q_ref, dp_ref, wn2_ref, out_ref):
    deg = _unfold_deg(dp_ref[...])
    q = q_ref[...]
    out_ref[...] = pre_ref[...] + jnp.dot(
        q[0] + q[1], wn2_ref[...], preferred_element_type=jnp.float32) / deg

  return pl.pallas_call(
      body,
      grid=(nb,),
      in_specs=[
          pl.BlockSpec((r, c_dim), lambda i: (i, 0)),
          pl.BlockSpec((2, r, h), lambda i: (0, i, 0)),
          pl.BlockSpec((2, r, 8), lambda i: (0, i, 0)),
          pl.BlockSpec((h, c_dim), lambda i: (0, 0)),
      ],
      out_specs=pl.BlockSpec((r, c_dim), lambda i: (i, 0)),
      out_shape=jax.ShapeDtypeStruct((n, c_dim), jnp.float32),
  )(pre2, sums2_p, deg_p, W_neigh2)


@jax.jit
def kernel(features, edge_index, W_self1, W_neigh1, b1, W_self2, W_neigh2,
           b2):
  n = features.shape[0]
  e = edge_index.shape[1]
  kpc = _NW * _K
  ch = -(-(-(-e // kpc)) // _BLK) * _BLK
  e_pad = ch * kpc
  src = edge_index[0]
  dst = edge_index[1]
  if e_pad != e:
    src = jnp.concatenate([src, jnp.zeros((e_pad - e,), jnp.int32)])
    dst = jnp.concatenate([dst, jnp.full((e_pad - e,), n, jnp.int32)])
  pattern = jnp.tile(
      jnp.kron(jnp.eye(16, dtype=jnp.float32),
               jnp.ones((1, 8), jnp.float32)), (128, 1))
  pat_idx = (dst & 15) + 16 * (jnp.arange(e_pad, dtype=jnp.int32) & 127)
  shape3 = (_NW, ch, 1, _K)
  srcs = src.reshape(shape3)
  dsts = dst.reshape(shape3)
  divs = (dst >> 4).reshape(shape3)
  pats = pat_idx.reshape(shape3)
  np_rows = -(-n // (_NS * _K)) * (_NS * _K)

  sums1_p, deg_p = _sc_aggregate(features, np_rows, srcs, dsts,
                                 divs=divs, pats=pats, pattern=pattern)
  deg_p = deg_p.reshape(_NC, np_rows, 8)
  h1, pre2 = _tc_layer1(features, sums1_p, deg_p, W_self1, W_neigh1, b1,
                        W_self2, b2)
  (sums2_p,) = _sc_aggregate(h1, np_rows, srcs, dsts)
  return _tc_layer2(pre2, sums2_p, deg_p, W_neigh2)


# speedup vs baseline: 3.2496x; 1.1093x over previous
import functools

import jax
import jax.numpy as jnp
from jax import lax
from jax.experimental import pallas as pl
from jax.experimental.pallas import tpu as pltpu
from jax.experimental.pallas import tpu_sc as plsc

_NC = 2
_NS = 16
_NW = _NC * _NS
_K = 80
_BLK = 25
_NBUF = 3


def _zero_buf(buf, k, width):
  def zrow(i, _):
    for cc in range(width // 16):
      buf[0, i, pl.ds(cc * 16, 16)] = jnp.zeros((16,), jnp.float32)
    return 0
  lax.fori_loop(0, k, zrow, 0)


def _sc_aggregate(table, n_rows, srcs, dsts):
  width = table.shape[1]
  nw, ch, _, k = srcs.shape
  npt = n_rows // _NS
  nblk = ch // _BLK
  mesh = plsc.VectorSubcoreMesh(core_axis_name="c", subcore_axis_name="s")

  def body(table_hbm, src_hbm, dst_hbm, sums_out, src_b, dst_b, rows_v,
           sums_sh, sem_g, sem_s):
    c = lax.axis_index("c")
    s = lax.axis_index("s")
    wid = c * _NS + s

    _zero_buf(rows_v, k, width)
    base = s * npt
    for t in range(npt // k):
      pltpu.sync_copy(rows_v.at[0], sums_sh.at[pl.ds(base + t * k, k)])
    plsc.subcore_barrier()

    def mk_g(j, b):
      return pltpu.make_async_copy(
          table_hbm.at[src_b.at[j, 0]], rows_v.at[b], sem_g)

    def mk_s(j, b):
      return pltpu.make_async_copy(
          rows_v.at[b], sums_sh.at[dst_b.at[j, 0]], sem_s)

    def block_step(bi, _):
      sl = pl.ds(bi * _BLK, _BLK)
      pltpu.sync_copy(src_hbm.at[wid, sl], src_b)
      pltpu.sync_copy(dst_hbm.at[wid, sl], dst_b)
      for i in range(_NBUF - 1):
        mk_g(i, i).start()

      def step(j, _):
        @pl.when(j >= 1)
        def _():
          mk_s(j - 1, (j - 1) % _NBUF).wait()

        @pl.when(j < _BLK - _NBUF + 1)
        def _():
          mk_g(j + _NBUF - 1, (j + _NBUF - 1) % _NBUF).start()
        mk_g(j, j % _NBUF).wait()
        mk_s(j, j % _NBUF).start(add=True)
        return 0
      lax.fori_loop(0, _BLK, step, 0)
      mk_s(_BLK - 1, (_BLK - 1) % _NBUF).wait()
      return 0
    lax.fori_loop(0, nblk, block_step, 0)

    plsc.subcore_barrier()
    pltpu.sync_copy(sums_sh.at[pl.ds(base, npt)],
                    sums_out.at[c, pl.ds(base, npt)])

  fn = pl.kernel(
      body,
      out_type=jax.ShapeDtypeStruct((_NC, n_rows, width), jnp.float32),
      mesh=mesh,
      scratch_types=[
          pltpu.VMEM((_BLK, 1, k), jnp.int32),
          pltpu.VMEM((_BLK, 1, k), jnp.int32),
          pltpu.VMEM((_NBUF, k, width), jnp.float32),
          pltpu.VMEM_SHARED((n_rows, width), jnp.float32),
          pltpu.SemaphoreType.DMA,
          pltpu.SemaphoreType.DMA,
      ])
  return fn(table, srcs, dsts)


def _sc_degree(n_rows, dsts):
  nw, ch, _, k = dsts.shape
  npt = n_rows // _NS
  nblk = ch // _BLK
  mesh = plsc.VectorSubcoreMesh(core_axis_name="c", subcore_axis_name="s")

  def body(dst_hbm, deg_out, dst_b, ones_v, deg_sh, sem_s):
    c = lax.axis_index("c")
    s = lax.axis_index("s")
    wid = c * _NS + s

    _zero_buf(ones_v, k, 128)
    base = s * npt
    for t in range(npt // k):
      pltpu.sync_copy(ones_v.at[0], deg_sh.at[pl.ds(base + t * k, k)])

    def orow(i, _):
      ones_v[0, i, pl.ds(0, 16)] = jnp.ones((16,), jnp.float32)
      return 0
    lax.fori_loop(0, k, orow, 0)
    plsc.subcore_barrier()

    def mk_s(j):
      return pltpu.make_async_copy(
          ones_v.at[0], deg_sh.at[dst_b.at[j, 0]], sem_s)

    def block_step(bi, _):
      pltpu.sync_copy(dst_hbm.at[wid, pl.ds(bi * _BLK, _BLK)], dst_b)

      def step(j, _):
        mk_s(j).start(add=True)

        @pl.when(j >= 3)
        def _():
          mk_s(j - 3).wait()
        return 0
      lax.fori_loop(0, _BLK, step, 0)
      for j in range(_BLK - 3, _BLK):
        mk_s(j).wait()
      return 0
    lax.fori_loop(0, nblk, block_step, 0)

    plsc.subcore_barrier()
    pltpu.sync_copy(deg_sh.at[pl.ds(base, npt)],
                    deg_out.at[c, pl.ds(base, npt)])

  fn = pl.kernel(
      body,
      out_type=jax.ShapeDtypeStruct((_NC, n_rows, 128), jnp.float32),
      mesh=mesh,
      scratch_types=[
          pltpu.VMEM((_BLK, 1, k), jnp.int32),
          pltpu.VMEM((1, k, 128), jnp.float32),
          pltpu.VMEM_SHARED((n_rows, 128), jnp.float32),
          pltpu.SemaphoreType.DMA,
      ])
  return fn(dsts)


def _tc_layer1(features, sums_p, deg_p, W_self1, W_neigh1, b1,
               W_self2, b2):
  n, d = features.shape
  h = W_self1.shape[1]
  c_dim = W_self2.shape[1]
  r = 1024
  nb = (n + r - 1) // r

  def body(f_ref, sp_ref, dp_ref, ws1_ref, wn1_ref, b1_ref, ws2_ref,
           b2_ref, h1_ref, pre2_ref):
    dp = dp_ref[...]
    deg = jnp.maximum(dp[0, :, :1] + dp[1, :, :1], 1.0)
    sp = sp_ref[...]
    sums1 = sp[0] + sp[1]
    hn1 = jnp.dot(sums1, wn1_ref[...],
                  preferred_element_type=jnp.float32) / deg
    h1 = jnp.maximum(
        jnp.dot(f_ref[...], ws1_ref[...], preferred_element_type=jnp.float32)
        + hn1 + b1_ref[...], 0.0)
    h1_ref[...] = h1
    pre2_ref[...] = (
        jnp.dot(h1, ws2_ref[...], preferred_element_type=jnp.float32)
        + b2_ref[...])

  return pl.pallas_call(
      body,
      grid=(nb,),
      in_specs=[
          pl.BlockSpec((r, d), lambda i: (i, 0)),
          pl.BlockSpec((2, r, d), lambda i: (0, i, 0)),
          pl.BlockSpec((2, r, 128), lambda i: (0, i, 0)),
          pl.BlockSpec((d, h), lambda i: (0, 0)),
          pl.BlockSpec((d, h), lambda i: (0, 0)),
          pl.BlockSpec((1, h), lambda i: (0, 0)),
          pl.BlockSpec((h, c_dim), lambda i: (0, 0)),
          pl.BlockSpec((1, c_dim), lambda i: (0, 0)),
      ],
      out_specs=[
          pl.BlockSpec((r, h), lambda i: (i, 0)),
          pl.BlockSpec((r, c_dim), lambda i: (i, 0)),
      ],
      out_shape=[
          jax.ShapeDtypeStruct((n, h), jnp.float32),
          jax.ShapeDtypeStruct((n, c_dim), jnp.float32),
      ],
  )(features, sums_p, deg_p, W_self1, W_neigh1, b1.reshape(1, h),
    W_self2, b2.reshape(1, c_dim))


def _tc_layer2(pre2, sums2_p, deg_p, W_neigh2):
  n, c_dim = pre2.shape
  h = W_neigh2.shape[0]
  r = 1024
  nb = (n + r - 1) // r

  def body(pre_ref, q_ref, dp_ref, wn2_ref, out_ref):
    dp = dp_ref[...]
    deg = jnp.maximum(dp[0, :, :1] + dp[1, :, :1], 1.0)
    q = q_ref[...]
    out_ref[...] = pre_ref[...] + jnp.dot(
        q[0] + q[1], wn2_ref[...], preferred_element_type=jnp.float32) / deg

  return pl.pallas_call(
      body,
      grid=(nb,),
      in_specs=[
          pl.BlockSpec((r, c_dim), lambda i: (i, 0)),
          pl.BlockSpec((2, r, h), lambda i: (0, i, 0)),
          pl.BlockSpec((2, r, 128), lambda i: (0, i, 0)),
          pl.BlockSpec((h, c_dim), lambda i: (0, 0)),
      ],
      out_specs=pl.BlockSpec((r, c_dim), lambda i: (i, 0)),
      out_shape=jax.ShapeDtypeStruct((n, c_dim), jnp.float32),
  )(pre2, sums2_p, deg_p, W_neigh2)


@jax.jit
def kernel(features, edge_index, W_self1, W_neigh1, b1, W_self2, W_neigh2,
           b2):
  n = features.shape[0]
  e = edge_index.shape[1]
  kpc = _NW * _K
  ch = -(-(-(-e // kpc)) // _BLK) * _BLK
  e_pad = ch * kpc
  src = edge_index[0]
  dst = edge_index[1]
  if e_pad != e:
    src = jnp.concatenate([src, jnp.zeros((e_pad - e,), jnp.int32)])
    dst = jnp.concatenate([dst, jnp.full((e_pad - e,), n, jnp.int32)])
  shape3 = (_NW, ch, 1, _K)
  srcs = src.reshape(shape3)
  dsts = dst.reshape(shape3)
  np_rows = -(-n // (_NS * _K)) * (_NS * _K)

  sums1_p = _sc_aggregate(features, np_rows, srcs, dsts)
  deg_p = _sc_degree(np_rows, dsts)
  h1, pre2 = _tc_layer1(features, sums1_p, deg_p, W_self1, W_neigh1, b1,
                        W_self2, b2)
  sums2_p = _sc_aggregate(h1, np_rows, srcs, dsts)
  return _tc_layer2(pre2, sums2_p, deg_p, W_neigh2)


# speedup vs baseline: 3.3957x; 1.0449x over previous
import functools

import jax
import jax.numpy as jnp
from jax import lax
from jax.experimental import pallas as pl
from jax.experimental.pallas import tpu as pltpu
from jax.experimental.pallas import tpu_sc as plsc

_NC = 2
_NS = 16
_NW = _NC * _NS
_K = 80
_BLK = 25
_NBUF = 3


def _zero_buf(buf, k, width):
  def zrow(i, _):
    for cc in range(width // 16):
      buf[0, i, pl.ds(cc * 16, 16)] = jnp.zeros((16,), jnp.float32)
    return 0
  lax.fori_loop(0, k, zrow, 0)


def _sc_aggregate(table, n_rows, srcs, dsts):
  width = table.shape[1]
  nw, ch, _, k = srcs.shape
  npt = n_rows // _NS
  nblk = ch // _BLK
  mesh = plsc.VectorSubcoreMesh(core_axis_name="c", subcore_axis_name="s")

  def body(table_hbm, src_hbm, dst_hbm, sums_out, src_b, dst_b, rows_v,
           sums_sh, sem_g, sem_s, sem_i):
    c = lax.axis_index("c")
    s = lax.axis_index("s")
    wid = c * _NS + s

    def mk_i(bi, ib):
      sl = pl.ds(bi * _BLK, _BLK)
      return (pltpu.make_async_copy(src_hbm.at[wid, sl], src_b.at[ib],
                                    sem_i),
              pltpu.make_async_copy(dst_hbm.at[wid, sl], dst_b.at[ib],
                                    sem_i))

    for d in mk_i(0, 0):
      d.start()
    _zero_buf(rows_v, k, width)
    base = s * npt
    for t in range(npt // k):
      pltpu.sync_copy(rows_v.at[0], sums_sh.at[pl.ds(base + t * k, k)])
    for d in mk_i(0, 0):
      d.wait()
    plsc.subcore_barrier()

    def block_step(bi, _):
      ib = bi % 2

      def mk_g(j, b):
        return pltpu.make_async_copy(
            table_hbm.at[src_b.at[ib, j, 0]], rows_v.at[b], sem_g)

      def mk_s(j, b):
        return pltpu.make_async_copy(
            rows_v.at[b], sums_sh.at[dst_b.at[ib, j, 0]], sem_s)

      @pl.when(bi + 1 < nblk)
      def _():
        for d in mk_i(bi + 1, 1 - ib):
          d.start()
      for i in range(_NBUF - 1):
        mk_g(i, i).start()

      def step(j, _):
        @pl.when(j >= 1)
        def _():
          mk_s(j - 1, (j - 1) % _NBUF).wait()

        @pl.when(j < _BLK - _NBUF + 1)
        def _():
          mk_g(j + _NBUF - 1, (j + _NBUF - 1) % _NBUF).start()
        mk_g(j, j % _NBUF).wait()
        mk_s(j, j % _NBUF).start(add=True)
        return 0
      lax.fori_loop(0, _BLK, step, 0)
      mk_s(_BLK - 1, (_BLK - 1) % _NBUF).wait()

      @pl.when(bi + 1 < nblk)
      def _():
        for d in mk_i(bi + 1, 1 - ib):
          d.wait()
      return 0
    lax.fori_loop(0, nblk, block_step, 0)

    plsc.subcore_barrier()
    pltpu.sync_copy(sums_sh.at[pl.ds(base, npt)],
                    sums_out.at[c, pl.ds(base, npt)])

  fn = pl.kernel(
      body,
      out_type=jax.ShapeDtypeStruct((_NC, n_rows, width), jnp.float32),
      mesh=mesh,
      scratch_types=[
          pltpu.VMEM((2, _BLK, 1, k), jnp.int32),
          pltpu.VMEM((2, _BLK, 1, k), jnp.int32),
          pltpu.VMEM((_NBUF, k, width), jnp.float32),
          pltpu.VMEM_SHARED((n_rows, width), jnp.float32),
          pltpu.SemaphoreType.DMA,
          pltpu.SemaphoreType.DMA,
          pltpu.SemaphoreType.DMA,
      ])
  return fn(table, srcs, dsts)


def _sc_degree(n_rows, dsts):
  nw, ch, _, k = dsts.shape
  npt = n_rows // _NS
  nblk = ch // _BLK
  mesh = plsc.VectorSubcoreMesh(core_axis_name="c", subcore_axis_name="s")

  def body(dst_hbm, deg_out, dst_b, ones_v, deg_sh, sem_s, sem_i):
    c = lax.axis_index("c")
    s = lax.axis_index("s")
    wid = c * _NS + s

    def mk_i(bi, ib):
      return pltpu.make_async_copy(
          dst_hbm.at[wid, pl.ds(bi * _BLK, _BLK)], dst_b.at[ib], sem_i)

    mk_i(0, 0).start()
    _zero_buf(ones_v, k, 128)
    base = s * npt
    for t in range(npt // k):
      pltpu.sync_copy(ones_v.at[0], deg_sh.at[pl.ds(base + t * k, k)])

    def orow(i, _):
      ones_v[0, i, pl.ds(0, 16)] = jnp.ones((16,), jnp.float32)
      return 0
    lax.fori_loop(0, k, orow, 0)
    mk_i(0, 0).wait()
    plsc.subcore_barrier()

    def block_step(bi, _):
      ib = bi % 2

      def mk_s(j):
        return pltpu.make_async_copy(
            ones_v.at[0], deg_sh.at[dst_b.at[ib, j, 0]], sem_s)

      @pl.when(bi + 1 < nblk)
      def _():
        mk_i(bi + 1, 1 - ib).start()

      def step(j, _):
        mk_s(j).start(add=True)

        @pl.when(j >= 6)
        def _():
          mk_s(j - 6).wait()
        return 0
      lax.fori_loop(0, _BLK, step, 0)
      for j in range(_BLK - 6, _BLK):
        mk_s(j).wait()

      @pl.when(bi + 1 < nblk)
      def _():
        mk_i(bi + 1, 1 - ib).wait()
      return 0
    lax.fori_loop(0, nblk, block_step, 0)

    plsc.subcore_barrier()
    pltpu.sync_copy(deg_sh.at[pl.ds(base, npt)],
                    deg_out.at[c, pl.ds(base, npt)])

  fn = pl.kernel(
      body,
      out_type=jax.ShapeDtypeStruct((_NC, n_rows, 128), jnp.float32),
      mesh=mesh,
      scratch_types=[
          pltpu.VMEM((2, _BLK, 1, k), jnp.int32),
          pltpu.VMEM((1, k, 128), jnp.float32),
          pltpu.VMEM_SHARED((n_rows, 128), jnp.float32),
          pltpu.SemaphoreType.DMA,
          pltpu.SemaphoreType.DMA,
      ])
  return fn(dsts)


def _tc_layer1(features, sums_p, deg_p, W_self1, W_neigh1, b1,
               W_self2, b2):
  n, d = features.shape
  h = W_self1.shape[1]
  c_dim = W_self2.shape[1]
  r = 1024
  nb = (n + r - 1) // r

  def body(f_ref, sp_ref, dp_ref, ws1_ref, wn1_ref, b1_ref, ws2_ref,
           b2_ref, h1_ref, pre2_ref):
    dp = dp_ref[...]
    deg = jnp.maximum(dp[0, :, :1] + dp[1, :, :1], 1.0)
    sp = sp_ref[...]
    sums1 = sp[0] + sp[1]
    hn1 = jnp.dot(sums1, wn1_ref[...],
                  preferred_element_type=jnp.float32) / deg
    h1 = jnp.maximum(
        jnp.dot(f_ref[...], ws1_ref[...], preferred_element_type=jnp.float32)
        + hn1 + b1_ref[...], 0.0)
    h1_ref[...] = h1
    pre2_ref[...] = (
        jnp.dot(h1, ws2_ref[...], preferred_element_type=jnp.float32)
        + b2_ref[...])

  return pl.pallas_call(
      body,
      grid=(nb,),
      in_specs=[
          pl.BlockSpec((r, d), lambda i: (i, 0)),
          pl.BlockSpec((2, r, d), lambda i: (0, i, 0)),
          pl.BlockSpec((2, r, 128), lambda i: (0, i, 0)),
          pl.BlockSpec((d, h), lambda i: (0, 0)),
          pl.BlockSpec((d, h), lambda i: (0, 0)),
          pl.BlockSpec((1, h), lambda i: (0, 0)),
          pl.BlockSpec((h, c_dim), lambda i: (0, 0)),
          pl.BlockSpec((1, c_dim), lambda i: (0, 0)),
      ],
      out_specs=[
          pl.BlockSpec((r, h), lambda i: (i, 0)),
          pl.BlockSpec((r, c_dim), lambda i: (i, 0)),
      ],
      out_shape=[
          jax.ShapeDtypeStruct((n, h), jnp.float32),
          jax.ShapeDtypeStruct((n, c_dim), jnp.float32),
      ],
  )(features, sums_p, deg_p, W_self1, W_neigh1, b1.reshape(1, h),
    W_self2, b2.reshape(1, c_dim))


def _tc_layer2(pre2, sums2_p, deg_p, W_neigh2):
  n, c_dim = pre2.shape
  h = W_neigh2.shape[0]
  r = 1024
  nb = (n + r - 1) // r

  def body(pre_ref, q_ref, dp_ref, wn2_ref, out_ref):
    dp = dp_ref[...]
    deg = jnp.maximum(dp[0, :, :1] + dp[1, :, :1], 1.0)
    q = q_ref[...]
    out_ref[...] = pre_ref[...] + jnp.dot(
        q[0] + q[1], wn2_ref[...], preferred_element_type=jnp.float32) / deg

  return pl.pallas_call(
      body,
      grid=(nb,),
      in_specs=[
          pl.BlockSpec((r, c_dim), lambda i: (i, 0)),
          pl.BlockSpec((2, r, h), lambda i: (0, i, 0)),
          pl.BlockSpec((2, r, 128), lambda i: (0, i, 0)),
          pl.BlockSpec((h, c_dim), lambda i: (0, 0)),
      ],
      out_specs=pl.BlockSpec((r, c_dim), lambda i: (i, 0)),
      out_shape=jax.ShapeDtypeStruct((n, c_dim), jnp.float32),
  )(pre2, sums2_p, deg_p, W_neigh2)


@jax.jit
def kernel(features, edge_index, W_self1, W_neigh1, b1, W_self2, W_neigh2,
           b2):
  n = features.shape[0]
  e = edge_index.shape[1]
  kpc = _NW * _K
  ch = -(-(-(-e // kpc)) // _BLK) * _BLK
  e_pad = ch * kpc
  src = edge_index[0]
  dst = edge_index[1]
  if e_pad != e:
    src = jnp.concatenate([src, jnp.zeros((e_pad - e,), jnp.int32)])
    dst = jnp.concatenate([dst, jnp.full((e_pad - e,), n, jnp.int32)])
  shape3 = (_NW, ch, 1, _K)
  srcs = src.reshape(shape3)
  dsts = dst.reshape(shape3)
  np_rows = -(-n // (_NS * _K)) * (_NS * _K)

  sums1_p = _sc_aggregate(features, np_rows, srcs, dsts)
  deg_p = _sc_degree(np_rows, dsts)
  h1, pre2 = _tc_layer1(features, sums1_p, deg_p, W_self1, W_neigh1, b1,
                        W_self2, b2)
  sums2_p = _sc_aggregate(h1, np_rows, srcs, dsts)
  return _tc_layer2(pre2, sums2_p, deg_p, W_neigh2)


# speedup vs baseline: 3.4830x; 1.0257x over previous
import functools

import jax
import jax.numpy as jnp
from jax import lax
from jax.experimental import pallas as pl
from jax.experimental.pallas import tpu as pltpu
from jax.experimental.pallas import tpu_sc as plsc

_NC = 2
_NS = 16
_NW = _NC * _NS
_K = 80
_BLK = 25
_NBUF = 3


def _zero_buf(buf, k, width):
  def zrow(i, _):
    for cc in range(width // 16):
      buf[0, i, pl.ds(cc * 16, 16)] = jnp.zeros((16,), jnp.float32)
    return 0
  lax.fori_loop(0, k, zrow, 0)


def _sc_aggregate(table, n_rows, srcs, dsts):
  width = table.shape[1]
  nw, nblk, _, k = srcs.shape
  npt = n_rows // _NS
  mesh = plsc.VectorSubcoreMesh(core_axis_name="c", subcore_axis_name="s")

  def body(table_hbm, src_hbm, dst_hbm, sums_out, src_b, dst_b, rows_v,
           sums_sh, sem_g, sem_s, sem_i):
    c = lax.axis_index("c")
    s = lax.axis_index("s")
    wid = c * _NS + s

    def mk_i(bi, ib):
      return (pltpu.make_async_copy(src_hbm.at[wid, bi], src_b.at[ib],
                                    sem_i),
              pltpu.make_async_copy(dst_hbm.at[wid, bi], dst_b.at[ib],
                                    sem_i))

    for d in mk_i(0, 0):
      d.start()
    _zero_buf(rows_v, k, width)
    base = s * npt
    for t in range(npt // k):
      pltpu.sync_copy(rows_v.at[0], sums_sh.at[pl.ds(base + t * k, k)])
    for d in mk_i(0, 0):
      d.wait()
    plsc.subcore_barrier()

    def block_step(bi, _):
      ib = bi % 2

      def mk_g(j, b):
        return pltpu.make_async_copy(
            table_hbm.at[src_b.at[ib, j]], rows_v.at[b], sem_g)

      def mk_s(j, b):
        return pltpu.make_async_copy(
            rows_v.at[b], sums_sh.at[dst_b.at[ib, j]], sem_s)

      @pl.when(bi + 1 < nblk)
      def _():
        for d in mk_i(bi + 1, 1 - ib):
          d.start()
      for i in range(_NBUF - 1):
        mk_g(i, i).start()

      def step(j, _):
        @pl.when(j >= 1)
        def _():
          mk_s(j - 1, (j - 1) % _NBUF).wait()

        @pl.when(j < _BLK - _NBUF + 1)
        def _():
          mk_g(j + _NBUF - 1, (j + _NBUF - 1) % _NBUF).start()
        mk_g(j, j % _NBUF).wait()
        mk_s(j, j % _NBUF).start(add=True)
        return 0
      lax.fori_loop(0, _BLK, step, 0)
      mk_s(_BLK - 1, (_BLK - 1) % _NBUF).wait()

      @pl.when(bi + 1 < nblk)
      def _():
        for d in mk_i(bi + 1, 1 - ib):
          d.wait()
      return 0
    lax.fori_loop(0, nblk, block_step, 0)

    plsc.subcore_barrier()
    pltpu.sync_copy(sums_sh.at[pl.ds(base, npt)],
                    sums_out.at[c, pl.ds(base, npt)])

  fn = pl.kernel(
      body,
      out_type=jax.ShapeDtypeStruct((_NC, n_rows, width), jnp.float32),
      mesh=mesh,
      scratch_types=[
          pltpu.VMEM((2, _BLK, k), jnp.int32),
          pltpu.VMEM((2, _BLK, k), jnp.int32),
          pltpu.VMEM((_NBUF, k, width), jnp.float32),
          pltpu.VMEM_SHARED((n_rows, width), jnp.float32),
          pltpu.SemaphoreType.DMA,
          pltpu.SemaphoreType.DMA,
          pltpu.SemaphoreType.DMA,
      ])
  return fn(table, srcs, dsts)


def _sc_degree(n_rows, dsts):
  nw, nblk, _, k = dsts.shape
  npt = n_rows // _NS
  mesh = plsc.VectorSubcoreMesh(core_axis_name="c", subcore_axis_name="s")

  def body(dst_hbm, deg_out, dst_b, ones_v, deg_sh, sem_s, sem_i):
    c = lax.axis_index("c")
    s = lax.axis_index("s")
    wid = c * _NS + s

    def mk_i(bi, ib):
      return pltpu.make_async_copy(
          dst_hbm.at[wid, bi], dst_b.at[ib], sem_i)

    mk_i(0, 0).start()
    _zero_buf(ones_v, k, 128)
    base = s * npt
    for t in range(npt // k):
      pltpu.sync_copy(ones_v.at[0], deg_sh.at[pl.ds(base + t * k, k)])

    def orow(i, _):
      ones_v[0, i, pl.ds(0, 16)] = jnp.ones((16,), jnp.float32)
      return 0
    lax.fori_loop(0, k, orow, 0)
    mk_i(0, 0).wait()
    plsc.subcore_barrier()

    def block_step(bi, _):
      ib = bi % 2

      def mk_s(j):
        return pltpu.make_async_copy(
            ones_v.at[0], deg_sh.at[dst_b.at[ib, j]], sem_s)

      @pl.when(bi + 1 < nblk)
      def _():
        mk_i(bi + 1, 1 - ib).start()

      def step(j, _):
        mk_s(j).start(add=True)

        @pl.when(j >= 6)
        def _():
          mk_s(j - 6).wait()
        return 0
      lax.fori_loop(0, _BLK, step, 0)
      for j in range(_BLK - 6, _BLK):
        mk_s(j).wait()

      @pl.when(bi + 1 < nblk)
      def _():
        mk_i(bi + 1, 1 - ib).wait()
      return 0
    lax.fori_loop(0, nblk, block_step, 0)

    plsc.subcore_barrier()
    pltpu.sync_copy(deg_sh.at[pl.ds(base, npt)],
                    deg_out.at[c, pl.ds(base, npt)])

  fn = pl.kernel(
      body,
      out_type=jax.ShapeDtypeStruct((_NC, n_rows, 128), jnp.float32),
      mesh=mesh,
      scratch_types=[
          pltpu.VMEM((2, _BLK, k), jnp.int32),
          pltpu.VMEM((1, k, 128), jnp.float32),
          pltpu.VMEM_SHARED((n_rows, 128), jnp.float32),
          pltpu.SemaphoreType.DMA,
          pltpu.SemaphoreType.DMA,
      ])
  return fn(dsts)


def _tc_layer1(features, sums_p, deg_p, W_self1, W_neigh1, b1,
               W_self2, b2):
  n, d = features.shape
  h = W_self1.shape[1]
  c_dim = W_self2.shape[1]
  r = 1024
  nb = (n + r - 1) // r

  def body(f_ref, sp_ref, dp_ref, ws1_ref, wn1_ref, b1_ref, ws2_ref,
           b2_ref, h1_ref, pre2_ref):
    dp = dp_ref[...]
    deg = jnp.maximum(dp[0, :, :1] + dp[1, :, :1], 1.0)
    sp = sp_ref[...]
    sums1 = sp[0] + sp[1]
    hn1 = jnp.dot(sums1, wn1_ref[...],
                  preferred_element_type=jnp.float32) / deg
    h1 = jnp.maximum(
        jnp.dot(f_ref[...], ws1_ref[...], preferred_element_type=jnp.float32)
        + hn1 + b1_ref[...], 0.0)
    h1_ref[...] = h1
    pre2_ref[...] = (
        jnp.dot(h1, ws2_ref[...], preferred_element_type=jnp.float32)
        + b2_ref[...])

  return pl.pallas_call(
      body,
      grid=(nb,),
      in_specs=[
          pl.BlockSpec((r, d), lambda i: (i, 0)),
          pl.BlockSpec((2, r, d), lambda i: (0, i, 0)),
          pl.BlockSpec((2, r, 128), lambda i: (0, i, 0)),
          pl.BlockSpec((d, h), lambda i: (0, 0)),
          pl.BlockSpec((d, h), lambda i: (0, 0)),
          pl.BlockSpec((1, h), lambda i: (0, 0)),
          pl.BlockSpec((h, c_dim), lambda i: (0, 0)),
          pl.BlockSpec((1, c_dim), lambda i: (0, 0)),
      ],
      out_specs=[
          pl.BlockSpec((r, h), lambda i: (i, 0)),
          pl.BlockSpec((r, c_dim), lambda i: (i, 0)),
      ],
      out_shape=[
          jax.ShapeDtypeStruct((n, h), jnp.float32),
          jax.ShapeDtypeStruct((n, c_dim), jnp.float32),
      ],
  )(features, sums_p, deg_p, W_self1, W_neigh1, b1.reshape(1, h),
    W_self2, b2.reshape(1, c_dim))


def _tc_layer2(pre2, sums2_p, deg_p, W_neigh2):
  n, c_dim = pre2.shape
  h = W_neigh2.shape[0]
  r = 1024
  nb = (n + r - 1) // r

  def body(pre_ref, q_ref, dp_ref, wn2_ref, out_ref):
    dp = dp_ref[...]
    deg = jnp.maximum(dp[0, :, :1] + dp[1, :, :1], 1.0)
    q = q_ref[...]
    out_ref[...] = pre_ref[...] + jnp.dot(
        q[0] + q[1], wn2_ref[...], preferred_element_type=jnp.float32) / deg

  return pl.pallas_call(
      body,
      grid=(nb,),
      in_specs=[
          pl.BlockSpec((r, c_dim), lambda i: (i, 0)),
          pl.BlockSpec((2, r, h), lambda i: (0, i, 0)),
          pl.BlockSpec((2, r, 128), lambda i: (0, i, 0)),
          pl.BlockSpec((h, c_dim), lambda i: (0, 0)),
      ],
      out_specs=pl.BlockSpec((r, c_dim), lambda i: (i, 0)),
      out_shape=jax.ShapeDtypeStruct((n, c_dim), jnp.float32),
  )(pre2, sums2_p, deg_p, W_neigh2)


@jax.jit
def kernel(features, edge_index, W_self1, W_neigh1, b1, W_self2, W_neigh2,
           b2):
  n = features.shape[0]
  e = edge_index.shape[1]
  kpc = _NW * _K
  ch = -(-(-(-e // kpc)) // _BLK) * _BLK
  e_pad = ch * kpc
  src = edge_index[0]
  dst = edge_index[1]
  if e_pad != e:
    src = jnp.concatenate([src, jnp.zeros((e_pad - e,), jnp.int32)])
    dst = jnp.concatenate([dst, jnp.full((e_pad - e,), n, jnp.int32)])
  shape4 = (_NW, ch // _BLK, _BLK, _K)
  srcs = src.reshape(shape4)
  dsts = dst.reshape(shape4)
  np_rows = -(-n // (_NS * _K)) * (_NS * _K)

  sums1_p = _sc_aggregate(features, np_rows, srcs, dsts)
  deg_p = _sc_degree(np_rows, dsts)
  h1, pre2 = _tc_layer1(features, sums1_p, deg_p, W_self1, W_neigh1, b1,
                        W_self2, b2)
  sums2_p = _sc_aggregate(h1, np_rows, srcs, dsts)
  return _tc_layer2(pre2, sums2_p, deg_p, W_neigh2)
